# baseline (speedup 1.0000x reference)
import functools

import jax
import jax.numpy as jnp
from jax import lax
from jax.experimental import pallas as pl
from jax.experimental.pallas import tpu as pltpu
from jax.experimental.pallas import tpu_sc as plsc

NUM_FIELDS = 26
EMBED_DIM = 32
BATCH = 4096
OUT_DIM = NUM_FIELDS * EMBED_DIM
GRP = 16


@functools.cache
def _build():
    info = plsc.get_sparse_core_info()
    nw = info.num_cores * info.num_subcores
    bpw = BATCH // nw
    nc = info.num_cores

    mesh = plsc.VectorSubcoreMesh(core_axis_name="c", subcore_axis_name="s")

    @functools.partial(
        pl.kernel,
        mesh=mesh,
        out_type=jax.ShapeDtypeStruct((BATCH, OUT_DIM), jnp.float32),
        scratch_types=[
            pltpu.VMEM((NUM_FIELDS, bpw + 16), jnp.int32),
            pltpu.VMEM((2, GRP, 8, EMBED_DIM), jnp.float32),
            pltpu.VMEM((GRP, OUT_DIM), jnp.float32),
            pltpu.SemaphoreType.DMA,
            pltpu.SemaphoreType.DMA,
        ],
    )
    def k(idx_hbm, *args):
        tables = args[:NUM_FIELDS]
        out_hbm, idx_v, buf_v, out_v, sem0, sem1 = args[NUM_FIELDS:]
        sems = (sem0, sem1)
        wid = lax.axis_index("s") * nc + lax.axis_index("c")
        base = wid * bpw
        pltpu.sync_copy(idx_hbm.at[:, pl.ds(base, bpw)], idx_v.at[:, pl.ds(0, bpw)])

        def fire(i, g):
            vec = idx_v[i, pl.ds(g * GRP, 16)]
            starts = vec & jnp.int32(-8)
            rrs = vec & jnp.int32(7)
            copies = []
            for j in range(GRP):
                start = pl.multiple_of(starts[j], 8)
                copies.append(
                    pltpu.async_copy(
                        tables[i].at[pl.ds(start, 8), :],
                        buf_v.at[i % 2, j],
                        sems[i % 2],
                    )
                )
            return rrs, copies

        def extract(i, rrs, copies):
            for c in copies:
                c.wait()
            col = i * EMBED_DIM
            for j in range(GRP):
                rr = rrs[j]
                out_v[j, pl.ds(col, 16)] = buf_v[i % 2, j, rr, pl.ds(0, 16)]
                out_v[j, pl.ds(col + 16, 16)] = buf_v[i % 2, j, rr, pl.ds(16, 16)]

        def grp_body(g, _):
            prev = fire(0, g)
            for i in range(NUM_FIELDS):
                nxt = fire(i + 1, g) if i + 1 < NUM_FIELDS else None
                extract(i, *prev)
                prev = nxt
            row0 = pl.multiple_of(base + g * GRP, GRP)
            pltpu.sync_copy(out_v, out_hbm.at[pl.ds(row0, GRP), :])
            return 0

        lax.fori_loop(0, bpw // GRP, grp_body, 0)

    return k


def kernel(*args):
    feats = args[:NUM_FIELDS]
    tables = args[NUM_FIELDS:]
    idx = jnp.stack(feats)
    return _build()(idx, *tables)
